# baseline (speedup 1.0000x reference)
import functools

import jax
import jax.numpy as jnp
from jax import lax
from jax.experimental import pallas as pl
from jax.experimental.pallas import tpu as pltpu
from jax.experimental.pallas import tpu_sc as plsc

DIM = 128
LANES = 16
ND = DIM // LANES
NC = 2
NS = 16
NW = NC * NS
BATCH = 16384
SC_BATCH = 12288
B_PER_W = SC_BATCH // NW
CHUNK = 64
N_CHUNKS = B_PER_W // CHUNK


def _make_sc_kernel():
    mesh = plsc.VectorSubcoreMesh(core_axis_name="c", subcore_axis_name="s")

    row_t = pltpu.VMEM((CHUNK, DIM), jnp.float32)

    @functools.partial(
        pl.kernel,
        mesh=mesh,
        out_type=jax.ShapeDtypeStruct((NW, LANES), jnp.float32),
        scratch_types=(
            [pltpu.VMEM((6, B_PER_W), jnp.int32)]
            + [row_t] * 6
            + [row_t] * 6
            + [pltpu.VMEM((LANES,), jnp.float32),
               pltpu.SemaphoreType.DMA,
               pltpu.SemaphoreType.DMA,
               pltpu.VMEM_SHARED((1000, DIM), jnp.float32),
               pltpu.VMEM_SHARED((1000, DIM), jnp.float32)]
        ),
    )
    def dist_mult(idx_hbm, ent_hbm, rel_hbm, out_hbm, *scratch):
        idx_v = scratch[0]
        row_a = scratch[1:7]
        row_b = scratch[7:13]
        out_v, sem_a, sem_b = scratch[13], scratch[14], scratch[15]
        ent_s, rel_s = scratch[16], scratch[17]

        tables = (ent_s, rel_s, ent_s, ent_s, rel_s, ent_s)

        cid = lax.axis_index("c")
        sid = lax.axis_index("s")
        wid = sid * NC + cid
        base = wid * B_PER_W

        iota = jnp.arange(LANES, dtype=jnp.int32)
        rots = [((iota + k) & (LANES - 1))[:, None] for k in (8, 4, 2, 1)]
        dnums = lax.GatherDimensionNumbers(
            offset_dims=(), collapsed_slice_dims=(0,), start_index_map=(0,))

        def hsum(v):
            for r in rots:
                v = v + lax.gather(
                    v, r, dnums, slice_sizes=(1,),
                    mode=lax.GatherScatterMode.PROMISE_IN_BOUNDS)
            return v

        @pl.when(sid == 0)
        def _():
            pltpu.sync_copy(ent_hbm.at[pl.ds(0, 1000)], ent_s)
            pltpu.sync_copy(rel_hbm, rel_s)

        pltpu.sync_copy(idx_hbm.at[:, pl.ds(base, B_PER_W)], idx_v)
        plsc.subcore_barrier()

        def issue(ci, rows, sem):
            for j, (tab, r) in enumerate(zip(tables, rows)):
                ib = idx_v.at[j, pl.ds(ci * CHUNK, CHUNK)]
                pltpu.async_copy(tab.at[ib], r, sem)

        def drain(ci, rows, sem):
            for j, (tab, r) in enumerate(zip(tables, rows)):
                ib = idx_v.at[j, pl.ds(ci * CHUNK, CHUNK)]
                pltpu.make_async_copy(tab.at[ib], r, sem).wait()

        def compute(rows, tot):
            hp_v, rp_v, tp_v, hn_v, rn_v, tn_v = rows

            def pair_body(i, t):
                s0 = pl.ds(0, LANES)
                accp = hp_v[i, s0] * rp_v[i, s0] * tp_v[i, s0]
                accn = hn_v[i, s0] * rn_v[i, s0] * tn_v[i, s0]
                for d in range(1, ND):
                    s = pl.ds(d * LANES, LANES)
                    accp = accp + hp_v[i, s] * rp_v[i, s] * tp_v[i, s]
                    accn = accn + hn_v[i, s] * rn_v[i, s] * tn_v[i, s]
                diff = hsum(accp - accn)
                return t + jnp.maximum(diff + 1.0, 0.0)

            return lax.fori_loop(0, CHUNK, pair_body, tot)

        issue(0, row_a, sem_a)

        def body(k, tot):
            issue(2 * k + 1, row_b, sem_b)
            drain(2 * k, row_a, sem_a)
            tot = compute(row_a, tot)

            nxt = 2 * k + 2

            @pl.when(nxt < N_CHUNKS)
            def _():
                issue(nxt, row_a, sem_a)

            drain(2 * k + 1, row_b, sem_b)
            return compute(row_b, tot)

        total = lax.fori_loop(0, N_CHUNKS // 2, body,
                              jnp.zeros((LANES,), jnp.float32))
        out_v[...] = total
        pltpu.sync_copy(out_v, out_hbm.at[wid])

    return dist_mult


_dist_mult = _make_sc_kernel()


@jax.jit
def kernel(positive_triples, negative_triples, entities, relations):
    pt = positive_triples.astype(jnp.int32)
    nt = negative_triples.astype(jnp.int32)
    idx_all = jnp.concatenate([pt[:SC_BATCH].T, nt[:SC_BATCH].T], axis=0)
    partials = _dist_mult(idx_all, entities, relations)
    sc_sum = jnp.sum(partials[:, 0])

    ent_c = entities[:1000]
    ptt = pt[SC_BATCH:]
    ntt = nt[SC_BATCH:]
    sp = jnp.sum(jnp.take(ent_c, ptt[:, 0], axis=0)
                 * jnp.take(relations, ptt[:, 1], axis=0)
                 * jnp.take(ent_c, ptt[:, 2], axis=0), axis=-1)
    sn = jnp.sum(jnp.take(ent_c, ntt[:, 0], axis=0)
                 * jnp.take(relations, ntt[:, 1], axis=0)
                 * jnp.take(ent_c, ntt[:, 2], axis=0), axis=-1)
    tc_sum = jnp.sum(jnp.maximum(sp - sn + 1.0, 0.0))
    return (sc_sum + tc_sum) / jnp.float32(BATCH)


# speedup vs baseline: 2.4465x; 2.4465x over previous
import functools

import jax
import jax.numpy as jnp
from jax import lax
from jax.experimental import pallas as pl
from jax.experimental.pallas import tpu as pltpu
from jax.experimental.pallas import tpu_sc as plsc

DIM = 128
LANES = 16
ND = DIM // LANES
NC = 2
NS = 16
NW = NC * NS
BATCH = 16384
B_PER_W = BATCH // NW
CHUNK = 64
N_CHUNKS = B_PER_W // CHUNK


def _make_sc_kernel():
    mesh = plsc.VectorSubcoreMesh(core_axis_name="c", subcore_axis_name="s")

    row_t = pltpu.VMEM((CHUNK, DIM), jnp.float32)

    @functools.partial(
        pl.kernel,
        mesh=mesh,
        out_type=jax.ShapeDtypeStruct((NW, LANES), jnp.float32),
        scratch_types=(
            [pltpu.VMEM((6, B_PER_W), jnp.int32)]
            + [row_t] * 6
            + [row_t] * 6
            + [pltpu.VMEM((LANES,), jnp.float32),
               pltpu.SemaphoreType.DMA,
               pltpu.SemaphoreType.DMA,
               pltpu.VMEM_SHARED((1000, DIM), jnp.float32),
               pltpu.VMEM_SHARED((1000, DIM), jnp.float32)]
        ),
    )
    def dist_mult(idx_hbm, ent_hbm, rel_hbm, out_hbm, *scratch):
        idx_v = scratch[0]
        row_a = scratch[1:7]
        row_b = scratch[7:13]
        out_v, sem_a, sem_b = scratch[13], scratch[14], scratch[15]
        ent_s, rel_s = scratch[16], scratch[17]

        tables = (ent_s, rel_s, ent_s, ent_s, rel_s, ent_s)

        cid = lax.axis_index("c")
        sid = lax.axis_index("s")
        wid = sid * NC + cid
        base = wid * B_PER_W

        iota = jnp.arange(LANES, dtype=jnp.int32)
        rots = [((iota + k) & (LANES - 1))[:, None] for k in (8, 4, 2, 1)]
        dnums = lax.GatherDimensionNumbers(
            offset_dims=(), collapsed_slice_dims=(0,), start_index_map=(0,))

        def hsum(v):
            for r in rots:
                v = v + lax.gather(
                    v, r, dnums, slice_sizes=(1,),
                    mode=lax.GatherScatterMode.PROMISE_IN_BOUNDS)
            return v

        @pl.when(sid == 0)
        def _():
            pltpu.sync_copy(ent_hbm.at[pl.ds(0, 1000)], ent_s)
            pltpu.sync_copy(rel_hbm, rel_s)

        pltpu.sync_copy(idx_hbm.at[:, pl.ds(base, B_PER_W)], idx_v)
        plsc.subcore_barrier()

        def issue(ci, rows, sem):
            for j, (tab, r) in enumerate(zip(tables, rows)):
                ib = idx_v.at[j, pl.ds(ci * CHUNK, CHUNK)]
                pltpu.async_copy(tab.at[ib], r, sem)

        def drain(ci, rows, sem):
            for j, (tab, r) in enumerate(zip(tables, rows)):
                ib = idx_v.at[j, pl.ds(ci * CHUNK, CHUNK)]
                pltpu.make_async_copy(tab.at[ib], r, sem).wait()

        def compute(rows, tot):
            hp_v, rp_v, tp_v, hn_v, rn_v, tn_v = rows

            def pair_body(i, t):
                s0 = pl.ds(0, LANES)
                accp = hp_v[i, s0] * rp_v[i, s0] * tp_v[i, s0]
                accn = hn_v[i, s0] * rn_v[i, s0] * tn_v[i, s0]
                for d in range(1, ND):
                    s = pl.ds(d * LANES, LANES)
                    accp = accp + hp_v[i, s] * rp_v[i, s] * tp_v[i, s]
                    accn = accn + hn_v[i, s] * rn_v[i, s] * tn_v[i, s]
                diff = hsum(accp - accn)
                return t + jnp.maximum(diff + 1.0, 0.0)

            return lax.fori_loop(0, CHUNK, pair_body, tot)

        issue(0, row_a, sem_a)

        def body(k, tot):
            issue(2 * k + 1, row_b, sem_b)
            drain(2 * k, row_a, sem_a)
            tot = compute(row_a, tot)

            nxt = 2 * k + 2

            @pl.when(nxt < N_CHUNKS)
            def _():
                issue(nxt, row_a, sem_a)

            drain(2 * k + 1, row_b, sem_b)
            return compute(row_b, tot)

        total = lax.fori_loop(0, N_CHUNKS // 2, body,
                              jnp.zeros((LANES,), jnp.float32))
        out_v[...] = total
        pltpu.sync_copy(out_v, out_hbm.at[wid])

    return dist_mult


_dist_mult = _make_sc_kernel()


@jax.jit
def kernel(positive_triples, negative_triples, entities, relations):
    pt = positive_triples.astype(jnp.int32)
    nt = negative_triples.astype(jnp.int32)
    idx_all = jnp.concatenate([pt.T, nt.T], axis=0)
    partials = _dist_mult(idx_all, entities, relations)
    return jnp.sum(partials[:, 0]) / jnp.float32(BATCH)
